# baseline (speedup 1.0000x reference)
import functools

import jax
import jax.numpy as jnp
from jax import lax
from jax.experimental import pallas as pl
from jax.experimental.pallas import tpu as pltpu
from jax.experimental.pallas import tpu_sc as plsc

VOCAB = 64001
DIM = 128
HALF = DIM // 2
MAX_LEN = 200
B = 4096
TOK = B * MAX_LEN
EPS = 1e-12


def _pack_bf16(x):
    tb = x.astype(jnp.bfloat16)
    a = lax.bitcast_convert_type(tb[:, :HALF], jnp.uint16).astype(jnp.uint32)
    b = lax.bitcast_convert_type(tb[:, HALF:], jnp.uint16).astype(jnp.uint32)
    return lax.bitcast_convert_type((b << 16) | a, jnp.int32)


def _transform_body(w_ref, w2_ref, o_ref):
    t = lax.dot_general(
        w_ref[...], w2_ref[...], (((1,), (1,)), ((), ())),
        preferred_element_type=jnp.float32)
    o_ref[...] = _pack_bf16(t)


def _transform_table(word_table, W2):
    R = 512
    return pl.pallas_call(
        _transform_body,
        grid=(pl.cdiv(VOCAB, R),),
        in_specs=[pl.BlockSpec((R, DIM), lambda i: (i, 0)),
                  pl.BlockSpec((DIM, DIM), lambda i: (0, 0))],
        out_specs=pl.BlockSpec((R, HALF), lambda i: (i, 0)),
        out_shape=jax.ShapeDtypeStruct((VOCAB, HALF), jnp.int32),
    )(word_table, W2)


def _pt_body(pos_ref, typ_ref, o_ref):
    x = pos_ref[...][:, None, :] + typ_ref[...][None, :, :]
    o_ref[...] = _pack_bf16(x.reshape(MAX_LEN * 3, DIM))


def _pt_table(pos_table, type_table):
    return pl.pallas_call(
        _pt_body,
        grid=(1,),
        in_specs=[pl.BlockSpec((MAX_LEN, DIM), lambda i: (0, 0)),
                  pl.BlockSpec((3, DIM), lambda i: (0, 0))],
        out_specs=pl.BlockSpec((MAX_LEN * 3, HALF), lambda i: (0, 0)),
        out_shape=jax.ShapeDtypeStruct((MAX_LEN * 3, HALF), jnp.int32),
    )(pos_table, type_table)


_NW = 32
_B_PER_W = TOK // _NW
_CHT = 160
_NCH = _B_PER_W // _CHT
_MASK_HI = -65536


def _sc_fused(tpk, ptk, ids3, seg3, gamma2, beta2):
    mesh = plsc.VectorSubcoreMesh(core_axis_name="c", subcore_axis_name="s")

    @functools.partial(
        pl.kernel,
        out_type=jax.ShapeDtypeStruct((TOK, DIM), jnp.float32),
        mesh=mesh,
        scratch_types=[
            pltpu.VMEM((2, 2, 80), jnp.int32),
            pltpu.VMEM((2, 2, 80), jnp.int32),
            pltpu.VMEM((2, 2, 80), jnp.int32),
            pltpu.VMEM((2, _CHT, HALF), jnp.int32),
            pltpu.VMEM((2, _CHT, HALF), jnp.int32),
            pltpu.VMEM((2, _CHT, DIM), jnp.float32),
            pltpu.VMEM((DIM,), jnp.float32),
            pltpu.VMEM((DIM,), jnp.float32),
            pltpu.SemaphoreType.DMA,
            pltpu.SemaphoreType.DMA,
            pltpu.SemaphoreType.DMA,
            pltpu.SemaphoreType.DMA,
        ],
        compiler_params=pltpu.CompilerParams(use_tc_tiling_on_sc=False,
                                             needs_layout_passes=False),
    )
    def k(tp_hbm, pt_hbm, ids_hbm, seg_hbm, gam_hbm, bet_hbm, out_hbm,
          bids, bseg, bpti, bT, bPT, bout, vgam, vbet,
          semg0, semg1, semo0, semo1):
        wid = lax.axis_index("s") * 2 + lax.axis_index("c")
        semg = (semg0, semg1)
        semo = (semo0, semo1)
        iota16 = lax.broadcasted_iota(jnp.int32, (16,), 0)

        def issue(g, b):
            irow = wid * (2 * _NCH) + g * 2
            pltpu.sync_copy(ids_hbm.at[pl.ds(irow, 2)], bids.at[b])
            pltpu.sync_copy(seg_hbm.at[pl.ds(irow, 2)], bseg.at[b])
            lbase = g * _CHT
            for r in range(2):
                for kk in range(5):
                    sl = bseg[b, r, pl.ds(kk * 16, 16)]
                    lv = lax.rem(lbase + r * 80 + kk * 16 + iota16,
                                 jnp.int32(MAX_LEN))
                    bpti[b, r, pl.ds(kk * 16, 16)] = lv * 3 + sl
            for r in range(2):
                pltpu.async_copy(tp_hbm.at[bids.at[b].at[r]],
                                 bT.at[b].at[pl.ds(r * 80, 80)], semg[b])
                pltpu.async_copy(pt_hbm.at[bpti.at[b].at[r]],
                                 bPT.at[b].at[pl.ds(r * 80, 80)], semg[b])

        def drain_gathers(b):
            for r in range(2):
                pltpu.make_async_copy(tp_hbm.at[bids.at[b].at[r]],
                                      bT.at[b].at[pl.ds(r * 80, 80)],
                                      semg[b]).wait()
                pltpu.make_async_copy(pt_hbm.at[bpti.at[b].at[r]],
                                      bPT.at[b].at[pl.ds(r * 80, 80)],
                                      semg[b]).wait()

        def compute(g, b):
            @pl.when(g >= 2)
            def _():
                pltpu.make_async_copy(bout.at[b],
                                      out_hbm.at[pl.ds(0, _CHT)],
                                      semo[b]).wait()
            drain_gathers(b)

            gb = tuple(vgam[pl.ds(j * 16, 16)] for j in range(8)) + \
                 tuple(vbet[pl.ds(j * 16, 16)] for j in range(8))

            def tok_body(t, carry):
                gv = carry[:8]
                bv = carry[8:]
                x = []
                for j in range(4):
                    wt = bT[b, t, pl.ds(j * 16, 16)]
                    wp = bPT[b, t, pl.ds(j * 16, 16)]
                    lo = plsc.bitcast(wt << 16, jnp.float32) + \
                         plsc.bitcast(wp << 16, jnp.float32)
                    hi = plsc.bitcast(wt & _MASK_HI, jnp.float32) + \
                         plsc.bitcast(wp & _MASK_HI, jnp.float32)
                    x.append((lo, hi))
                x8 = [p[0] for p in x] + [p[1] for p in x]
                s = x8[0]
                for v in x8[1:]:
                    s = s + v
                total = jnp.sum(s)
                sq = x8[0] * x8[0]
                for v in x8[1:]:
                    sq = sq + v * v
                total2 = jnp.sum(sq)
                mean = total * (1.0 / DIM)
                var = total2 * (1.0 / DIM) - mean * mean + EPS
                i0 = lax.bitcast_convert_type(var, jnp.int32)
                y0 = lax.bitcast_convert_type(
                    jnp.int32(0x5F3759DF) - (i0 >> 1), jnp.float32)
                h = var * 0.5
                y1 = y0 * (1.5 - h * y0 * y0)
                y2 = y1 * (1.5 - h * y1 * y1)
                r = y2 * (1.5 - h * y2 * y2)
                mv = jnp.broadcast_to(mean, (16,))
                rv = jnp.broadcast_to(r, (16,))
                for j in range(8):
                    yj = (x8[j] - mv) * rv * gv[j] + bv[j]
                    bout[b, t, pl.ds(j * 16, 16)] = yj
                return carry

            lax.fori_loop(0, _CHT, tok_body, gb, unroll=2)
            base = wid * _B_PER_W + g * _CHT
            pltpu.async_copy(bout.at[b], out_hbm.at[pl.ds(base, _CHT)],
                             semo[b])

        pltpu.sync_copy(gam_hbm, vgam)
        pltpu.sync_copy(bet_hbm, vbet)
        issue(0, 0)

        def body(i, carry):
            g0 = 2 * i
            issue(g0 + 1, 1)
            compute(g0, 0)

            @pl.when(g0 + 2 < _NCH)
            def _():
                issue(g0 + 2, 0)
            compute(g0 + 1, 1)
            return carry

        lax.fori_loop(0, _NCH // 2, body, 0)
        pltpu.make_async_copy(bout.at[0], out_hbm.at[pl.ds(0, _CHT)],
                              semo0).wait()
        pltpu.make_async_copy(bout.at[1], out_hbm.at[pl.ds(0, _CHT)],
                              semo1).wait()

    return k(tpk, ptk, ids3, seg3, gamma2, beta2)


def kernel(input_ids, segment_ids, word_table, W2, pos_table, type_table,
           gamma, beta):
    tpk = _transform_table(word_table, W2)
    ptk = _pt_table(pos_table, type_table)
    ids3 = input_ids.astype(jnp.int32).reshape(TOK // 80, 80)
    seg3 = segment_ids.astype(jnp.int32).reshape(TOK // 80, 80)
    out = _sc_fused(tpk, ptk, ids3, seg3, gamma, beta)
    return out.reshape(B, MAX_LEN, DIM)


# speedup vs baseline: 1.2491x; 1.2491x over previous
import functools

import jax
import jax.numpy as jnp
from jax import lax
from jax.experimental import pallas as pl
from jax.experimental.pallas import tpu as pltpu
from jax.experimental.pallas import tpu_sc as plsc

VOCAB = 64001
DIM = 128
HALF = DIM // 2
MAX_LEN = 200
B = 4096
TOK = B * MAX_LEN
EPS = 1e-12


def _transform_body(w_ref, w2_ref, o_ref):
    t = lax.dot_general(
        w_ref[...], w2_ref[...], (((1,), (1,)), ((), ())),
        preferred_element_type=jnp.float32)
    tb = t.astype(jnp.bfloat16)
    a = lax.bitcast_convert_type(tb[:, :HALF], jnp.uint16).astype(jnp.uint32)
    b = lax.bitcast_convert_type(tb[:, HALF:], jnp.uint16).astype(jnp.uint32)
    o_ref[...] = lax.bitcast_convert_type((b << 16) | a, jnp.int32)


def _transform_table(word_table, W2):
    R = 512
    return pl.pallas_call(
        _transform_body,
        grid=(pl.cdiv(VOCAB, R),),
        in_specs=[pl.BlockSpec((R, DIM), lambda i: (i, 0)),
                  pl.BlockSpec((DIM, DIM), lambda i: (0, 0))],
        out_specs=pl.BlockSpec((R, HALF), lambda i: (i, 0)),
        out_shape=jax.ShapeDtypeStruct((VOCAB, HALF), jnp.int32),
    )(word_table, W2)


_NW = 32
_B_PER_W = TOK // _NW
_CH = 512
_NCH = _B_PER_W // _CH


def _sc_gather(table, ids2d):
    mesh = plsc.VectorSubcoreMesh(core_axis_name="c", subcore_axis_name="s")

    @functools.partial(
        pl.kernel,
        out_type=jax.ShapeDtypeStruct((TOK, HALF), jnp.int32),
        mesh=mesh,
        scratch_types=[
            pltpu.VMEM((4, 128), jnp.int32),
            pltpu.VMEM((_CH, HALF), jnp.int32),
            pltpu.SemaphoreType.DMA,
        ],
        compiler_params=pltpu.CompilerParams(use_tc_tiling_on_sc=False),
    )
    def k(t_hbm, ids_hbm, out_hbm, idx_v, rows_v, sem):
        wid = lax.axis_index("s") * 2 + lax.axis_index("c")

        def body(g, carry):
            base = wid * _B_PER_W + g * _CH
            irow = wid * (_B_PER_W // 128) + g * (_CH // 128)
            pltpu.sync_copy(ids_hbm.at[pl.ds(irow, _CH // 128)], idx_v)
            cps = [
                pltpu.async_copy(t_hbm.at[idx_v.at[j]],
                                 rows_v.at[pl.ds(j * 128, 128)], sem)
                for j in range(_CH // 128)
            ]
            for c in cps:
                c.wait()
            pltpu.sync_copy(rows_v, out_hbm.at[pl.ds(base, _CH)])
            return carry

        lax.fori_loop(0, _NCH, body, 0)

    return k(table, ids2d)


_BR = 16
_N = _BR * MAX_LEN


def _ln_body(g_ref, seg_ref, poslo_ref, poshi_ref, typlo_ref, typhi_ref,
             gamlo_ref, gamhi_ref, betlo_ref, bethi_ref, o_ref):
    g = g_ref[...]
    lo = lax.bitcast_convert_type(g << 16, jnp.float32)
    hi = lax.bitcast_convert_type(g & jnp.int32(-65536), jnp.float32)
    seg = seg_ref[...].reshape(_N, 1)
    oneh = (seg == lax.broadcasted_iota(jnp.int32, (_N, 8), 1)
            ).astype(jnp.float32)
    tlo = lax.dot_general(oneh, typlo_ref[...], (((1,), (0,)), ((), ())),
                          preferred_element_type=jnp.float32)
    thi = lax.dot_general(oneh, typhi_ref[...], (((1,), (0,)), ((), ())),
                          preferred_element_type=jnp.float32)
    xlo = (lo + poslo_ref[...][None, :, :]).reshape(_N, HALF) + tlo
    xhi = (hi + poshi_ref[...][None, :, :]).reshape(_N, HALF) + thi
    ones = jnp.ones((HALF, 1), jnp.float32)
    dot = lambda a: lax.dot_general(a, ones, (((1,), (0,)), ((), ())),
                                    preferred_element_type=jnp.float32)
    ssum = dot(xlo) + dot(xhi)
    ssq = dot(xlo * xlo) + dot(xhi * xhi)
    mean = ssum * (1.0 / DIM)
    var = ssq * (1.0 / DIM) - mean * mean
    r = lax.rsqrt(var + EPS)
    ylo = (xlo - mean) * r * gamlo_ref[...] + betlo_ref[...]
    yhi = (xhi - mean) * r * gamhi_ref[...] + bethi_ref[...]
    y = jnp.concatenate([ylo.reshape(_BR, MAX_LEN, HALF),
                         yhi.reshape(_BR, MAX_LEN, HALF)], axis=-1)
    o_ref[...] = y


def _ln(gathered, segment_ids, pos_table, type_table, gamma, beta):
    full = lambda shape: pl.BlockSpec(shape, lambda i: tuple(0 for _ in shape))
    typ8 = jnp.zeros((8, DIM), jnp.float32).at[:3].set(type_table)
    return pl.pallas_call(
        _ln_body,
        grid=(B // _BR,),
        in_specs=[
            pl.BlockSpec((_BR, MAX_LEN, HALF), lambda i: (i, 0, 0)),
            pl.BlockSpec((_BR, MAX_LEN, 1), lambda i: (i, 0, 0)),
            full((MAX_LEN, HALF)), full((MAX_LEN, HALF)),
            full((8, HALF)), full((8, HALF)),
            full((1, HALF)), full((1, HALF)),
            full((1, HALF)), full((1, HALF)),
        ],
        out_specs=pl.BlockSpec((_BR, MAX_LEN, DIM), lambda i: (i, 0, 0)),
        out_shape=jax.ShapeDtypeStruct((B, MAX_LEN, DIM), jnp.float32),
    )(gathered, segment_ids.reshape(B, MAX_LEN, 1),
      pos_table[:, :HALF], pos_table[:, HALF:],
      typ8[:, :HALF], typ8[:, HALF:],
      gamma[:HALF].reshape(1, HALF), gamma[HALF:].reshape(1, HALF),
      beta[:HALF].reshape(1, HALF), beta[HALF:].reshape(1, HALF))


def kernel(input_ids, segment_ids, word_table, W2, pos_table, type_table,
           gamma, beta):
    table = _transform_table(word_table, W2)
    ids2d = input_ids.astype(jnp.int32).reshape(TOK // 128, 128)
    gathered = _sc_gather(table, ids2d)
    return _ln(gathered.reshape(B, MAX_LEN, HALF), segment_ids.astype(jnp.int32),
               pos_table, type_table, gamma, beta)
